# baseline (speedup 1.0000x reference)
import functools

import jax
import jax.numpy as jnp
import numpy as np
from jax import lax
from jax.experimental import pallas as pl
from jax.experimental.pallas import tpu as pltpu
from jax.experimental.pallas import tpu_sc as plsc

CH = 8
GRID = 104
STEP = np.float32(0.01)
ORIGIN = np.float32(0.0) - STEP * np.float32(1.0 + 1e-8)
MAX_COORD = np.float32(102.0)

L = 16
NW = 32
G = 196
PT = G * L
NPAD = NW * PT


def _coef_list(x):
    omx = 1.0 - x
    c0 = omx * omx * omx
    c2 = (omx * omx) * (x + 1.0) * (-3.0) + 4.0
    c1 = (x - 2.0) * (x * x) * 3.0 + 4.0
    c3 = x * x * x
    return [c0 / 6.0, c1 / 6.0, c2 / 6.0, c3 / 6.0]


NCELL = GRID * GRID * GRID
CPT = NCELL // NW
TCH = 2048
TLAST = CPT - TCH


NTCH = CPT // TCH + 1


def _tr_body(cp, table, in_v, out_v, sem0, sem1):
    c = lax.axis_index("c")
    s = lax.axis_index("s")
    wid = s * 2 + c
    cellstart = wid * CPT

    iota = lax.iota(jnp.int32, L)
    sems = [sem0, sem1]

    def chunk_start(j):
        return cellstart + jnp.minimum(j * TCH, TLAST)

    def make_in(j, slot):
        start = chunk_start(j)
        return [
            pltpu.make_async_copy(
                cp.at[pl.ds(ch * NCELL + start, TCH)],
                in_v.at[pl.ds(slot * (CH * TCH) + ch * TCH, TCH)], sems[slot])
            for ch in range(CH)
        ]

    def fire(j, slot):
        for cpd in make_in(j, slot):
            cpd.start()

    def consume(j, slot):
        for cpd in make_in(j, slot):
            cpd.wait()
        ibase = slot * (CH * TCH)

        col4 = [jnp.full((L,), cc, jnp.int32) for cc in range(CH // 2)]

        def inter(v2, carry2):
            for u in range(2):
                v = v2 * 2 + u
                rows = iota + v * L
                for cc in range(CH // 2):
                    xa = in_v[pl.ds(ibase + (2 * cc) * TCH + v * L, L)]
                    xb = in_v[pl.ds(ibase + (2 * cc + 1) * TCH + v * L, L)]
                    pk = plsc.pack(xa, xb, format=plsc.PackFormat.INTERLEAVED)
                    plsc.store_scatter(out_v, [rows, col4[cc]],
                                       plsc.bitcast(pk, jnp.int32))
            return carry2

        lax.fori_loop(0, TCH // L // 2, inter, 0)
        pltpu.sync_copy(out_v, table.at[pl.ds(chunk_start(j), TCH)])

    fire(0, 0)

    def body(jj, carry):
        j = jj * 2
        fire(j + 1, 1)
        consume(j, 0)

        @pl.when(jj < NTCH // 2 - 1)
        def _():
            fire(j + 2, 0)

        consume(j + 1, 1)
        return carry

    lax.fori_loop(0, NTCH // 2, body, 0)


def _sc_body(table, pts_t, pad, out, pts_v, idx_v, rows_v, out_v, pad_v,
             sem0, sem1):
    c = lax.axis_index("c")
    s = lax.axis_index("s")
    wid = s * 2 + c

    pltpu.sync_copy(pts_t.at[wid], pts_v)
    pltpu.sync_copy(pad, pad_v)

    iota = lax.iota(jnp.int32, L)
    iota3 = iota * 3
    iota8 = iota * CH
    padv = pad_v[...]
    zero = jnp.zeros((L,), jnp.float32)
    one = jnp.full((L,), 1.0, jnp.float32)
    col_idx = [jnp.full((L,), cp4, jnp.int32) for cp4 in range(CH // 2)]
    sems = [sem0, sem1]

    def make_cps(slot):
        return [
            pltpu.make_async_copy(table.at[idx_v.at[slot * 8 + j]],
                                  rows_v.at[pl.ds(slot * 1024 + j * 128, 128)],
                                  sems[slot])
            for j in range(8)
        ]

    def load_p(g):
        base = g * (3 * L)
        p0 = plsc.load_gather(pts_v, [iota3 + base])
        p1 = plsc.load_gather(pts_v, [iota3 + (base + 1)])
        p2 = plsc.load_gather(pts_v, [iota3 + (base + 2)])
        p0 = (p0 - ORIGIN) / STEP
        p1 = (p1 - ORIGIN) / STEP
        p2 = (p2 - ORIGIN) / STEP
        inb = ((p0 >= 1.0) & (p1 >= 1.0) & (p2 >= 1.0)
               & (p0 < MAX_COORD) & (p1 < MAX_COORD) & (p2 < MAX_COORD))
        p0 = jnp.where(inb, p0, one)
        p1 = jnp.where(inb, p1, one)
        p2 = jnp.where(inb, p2, one)
        return p0, p1, p2, inb

    def fire(g, slot):
        p0, p1, p2, _ = load_p(g)
        i0 = p0.astype(jnp.int32)
        i1 = p1.astype(jnp.int32)
        i2 = p2.astype(jnp.int32)
        base = (i0 * GRID + i1) * GRID + i2
        for a in range(4):
            for b in range(4):
                for cc in range(4):
                    k = a * 16 + b * 4 + cc
                    off = (a - 1) * GRID * GRID + (b - 1) * GRID + (cc - 1)
                    idx_v[slot * 8 + k // 8, pl.ds((k % 8) * L, L)] = base + off
        for cp in make_cps(slot):
            cp.start()

    def consume(g, slot):
        p0, p1, p2, inb = load_p(g)
        i0 = p0.astype(jnp.int32)
        i1 = p1.astype(jnp.int32)
        i2 = p2.astype(jnp.int32)
        c0s = _coef_list(p0 - i0.astype(jnp.float32))
        c1s = _coef_list(p1 - i1.astype(jnp.float32))
        c2s = _coef_list(p2 - i2.astype(jnp.float32))

        for cp in make_cps(slot):
            cp.wait()

        acc = [zero for _ in range(CH)]
        for a in range(4):
            for b in range(4):
                wab = c0s[a] * c1s[b]
                for cc in range(4):
                    k = a * 16 + b * 4 + cc
                    w = wab * c2s[cc]
                    ridx = iota + (slot * 1024 + k * L)
                    for cp4 in range(CH // 2):
                        vi = plsc.load_gather(rows_v, [ridx, col_idx[cp4]])
                        vb = plsc.bitcast(vi, jnp.bfloat16)
                        va, vc = plsc.unpack(
                            vb, format=plsc.PackFormat.INTERLEAVED)
                        acc[2 * cp4] = acc[2 * cp4] + w * va
                        acc[2 * cp4 + 1] = acc[2 * cp4 + 1] + w * vc

        base_o = g * (CH * L)
        for ch in range(CH):
            res = jnp.where(inb, acc[ch], padv)
            plsc.store_scatter(out_v, [iota8 + (base_o + ch)], res)

    fire(0, 0)

    def body(gg, carry):
        g = gg * 2
        fire(g + 1, 1)
        consume(g, 0)

        @pl.when(gg < G // 2 - 1)
        def _():
            fire(g + 2, 0)

        consume(g + 1, 1)
        return carry

    lax.fori_loop(0, G // 2, body, 0)
    pltpu.sync_copy(out_v, out.at[wid])


@jax.jit
def _run(cp_flat, pts_t, pad):
    mesh = plsc.VectorSubcoreMesh(core_axis_name="c", subcore_axis_name="s")
    tr = functools.partial(
        pl.kernel,
        out_type=jax.ShapeDtypeStruct((NCELL, CH // 2), jnp.int32),
        mesh=mesh,
        scratch_types=[
            pltpu.VMEM((2 * CH * TCH,), jnp.float32),
            pltpu.VMEM((TCH, CH // 2), jnp.int32),
            pltpu.SemaphoreType.DMA,
            pltpu.SemaphoreType.DMA,
        ],
        compiler_params=pltpu.CompilerParams(
            needs_layout_passes=False, use_tc_tiling_on_sc=False),
    )(_tr_body)
    table = tr(cp_flat)
    f = functools.partial(
        pl.kernel,
        out_type=jax.ShapeDtypeStruct((NW, PT * CH), jnp.float32),
        mesh=mesh,
        scratch_types=[
            pltpu.VMEM((PT * 3,), jnp.float32),
            pltpu.VMEM((16, 128), jnp.int32),
            pltpu.VMEM((2048, CH // 2), jnp.int32),
            pltpu.VMEM((PT * CH,), jnp.float32),
            pltpu.VMEM((L,), jnp.float32),
            pltpu.SemaphoreType.DMA,
            pltpu.SemaphoreType.DMA,
        ],
        compiler_params=pltpu.CompilerParams(
            needs_layout_passes=False, use_tc_tiling_on_sc=False),
    )(_sc_body)
    return f(table, pts_t, pad)


def kernel(pts, control_pts, padding_value):
    n = pts.shape[0]
    pts_pad = jnp.concatenate(
        [pts.astype(jnp.float32),
         jnp.full((NPAD - n, 3), 0.5, jnp.float32)], axis=0)
    pts_t = pts_pad.reshape(NW, PT * 3)
    cp_flat = control_pts.reshape(-1)
    pad = jnp.full((L,), padding_value, jnp.float32)
    out = _run(cp_flat, pts_t, pad)
    return out.reshape(NPAD, CH)[:n]


# speedup vs baseline: 1.1066x; 1.1066x over previous
import functools

import jax
import jax.numpy as jnp
import numpy as np
from jax import lax
from jax.experimental import pallas as pl
from jax.experimental.pallas import tpu as pltpu
from jax.experimental.pallas import tpu_sc as plsc

CH = 8
GRID = 104
STEP = np.float32(0.01)
ORIGIN = np.float32(0.0) - STEP * np.float32(1.0 + 1e-8)
MAX_COORD = np.float32(102.0)

L = 16
NW = 32
G = 196
PT = G * L
NPAD = NW * PT


def _coef_list(x):
    omx = 1.0 - x
    c0 = omx * omx * omx
    c2 = (omx * omx) * (x + 1.0) * (-3.0) + 4.0
    c1 = (x - 2.0) * (x * x) * 3.0 + 4.0
    c3 = x * x * x
    return [c0 / 6.0, c1 / 6.0, c2 / 6.0, c3 / 6.0]


NCELL = GRID * GRID * GRID
CPT = NCELL // NW
TCH = 2048
TLAST = CPT - TCH


NTCH = CPT // TCH + 1


def _tr_body(cp, table, in_v, out_v, sem0, sem1, osem0, osem1):
    c = lax.axis_index("c")
    s = lax.axis_index("s")
    wid = s * 2 + c
    cellstart = wid * CPT

    iota = lax.iota(jnp.int32, L)
    sems = [sem0, sem1]
    osems = [osem0, osem1]

    def chunk_start(j):
        return cellstart + jnp.minimum(j * TCH, TLAST)

    def make_out(j, slot):
        return pltpu.make_async_copy(
            out_v.at[pl.ds(slot * TCH, TCH)],
            table.at[pl.ds(chunk_start(j), TCH)], osems[slot])

    def make_in(j, slot):
        start = chunk_start(j)
        return [
            pltpu.make_async_copy(
                cp.at[pl.ds(ch * NCELL + start, TCH)],
                in_v.at[pl.ds(slot * (CH * TCH) + ch * TCH, TCH)], sems[slot])
            for ch in range(CH)
        ]

    def fire(j, slot):
        for cpd in make_in(j, slot):
            cpd.start()

    def consume(j, slot):
        for cpd in make_in(j, slot):
            cpd.wait()

        @pl.when(j >= 2)
        def _():
            make_out(j - 2, slot).wait()

        ibase = slot * (CH * TCH)

        col8 = [jnp.full((L,), ch, jnp.int32) for ch in range(CH)]
        obase = slot * TCH

        def inter(v2, carry2):
            for u in range(2):
                v = v2 * 2 + u
                rows = iota + (obase + v * L)
                for ch in range(CH):
                    x = in_v[pl.ds(ibase + ch * TCH + v * L, L)]
                    plsc.store_scatter(out_v, [rows, col8[ch]], x)
            return carry2

        lax.fori_loop(0, TCH // L // 2, inter, 0)
        make_out(j, slot).start()

    fire(0, 0)

    def body(jj, carry):
        j = jj * 2
        fire(j + 1, 1)
        consume(j, 0)

        @pl.when(jj < NTCH // 2 - 1)
        def _():
            fire(j + 2, 0)

        consume(j + 1, 1)
        return carry

    lax.fori_loop(0, NTCH // 2, body, 0)
    make_out(NTCH - 2, 0).wait()
    make_out(NTCH - 1, 1).wait()


def _sc_body(table, pts_t, pad, out, pts_v, idx_v, rows_v, out_v, pad_v,
             sem0, sem1):
    c = lax.axis_index("c")
    s = lax.axis_index("s")
    wid = s * 2 + c

    pltpu.sync_copy(pts_t.at[wid], pts_v)
    pltpu.sync_copy(pad, pad_v)

    iota = lax.iota(jnp.int32, L)
    iota3 = iota * 3
    iota8 = iota * CH
    padv = pad_v[...]
    zero = jnp.zeros((L,), jnp.float32)
    one = jnp.full((L,), 1.0, jnp.float32)
    col_idx = [jnp.full((L,), ch, jnp.int32) for ch in range(CH)]
    sems = [sem0, sem1]

    def make_cps(slot):
        return [
            pltpu.make_async_copy(table.at[idx_v.at[slot * 8 + j]],
                                  rows_v.at[pl.ds(slot * 1024 + j * 128, 128)],
                                  sems[slot])
            for j in range(8)
        ]

    def load_p(g):
        base = g * (3 * L)
        p0 = plsc.load_gather(pts_v, [iota3 + base])
        p1 = plsc.load_gather(pts_v, [iota3 + (base + 1)])
        p2 = plsc.load_gather(pts_v, [iota3 + (base + 2)])
        p0 = (p0 - ORIGIN) / STEP
        p1 = (p1 - ORIGIN) / STEP
        p2 = (p2 - ORIGIN) / STEP
        inb = ((p0 >= 1.0) & (p1 >= 1.0) & (p2 >= 1.0)
               & (p0 < MAX_COORD) & (p1 < MAX_COORD) & (p2 < MAX_COORD))
        p0 = jnp.where(inb, p0, one)
        p1 = jnp.where(inb, p1, one)
        p2 = jnp.where(inb, p2, one)
        return p0, p1, p2, inb

    def fire(g, slot):
        p0, p1, p2, _ = load_p(g)
        i0 = p0.astype(jnp.int32)
        i1 = p1.astype(jnp.int32)
        i2 = p2.astype(jnp.int32)
        base = (i0 * GRID + i1) * GRID + i2
        for a in range(4):
            for b in range(4):
                for cc in range(4):
                    k = a * 16 + b * 4 + cc
                    off = (a - 1) * GRID * GRID + (b - 1) * GRID + (cc - 1)
                    idx_v[slot * 8 + k // 8, pl.ds((k % 8) * L, L)] = base + off
        for cp in make_cps(slot):
            cp.start()

    def consume(g, slot):
        p0, p1, p2, inb = load_p(g)
        i0 = p0.astype(jnp.int32)
        i1 = p1.astype(jnp.int32)
        i2 = p2.astype(jnp.int32)
        c0s = _coef_list(p0 - i0.astype(jnp.float32))
        c1s = _coef_list(p1 - i1.astype(jnp.float32))
        c2s = _coef_list(p2 - i2.astype(jnp.float32))

        for cp in make_cps(slot):
            cp.wait()

        acc = [zero for _ in range(CH)]
        for a in range(4):
            for b in range(4):
                wab = c0s[a] * c1s[b]
                for cc in range(4):
                    k = a * 16 + b * 4 + cc
                    w = wab * c2s[cc]
                    ridx = iota + (slot * 1024 + k * L)
                    for ch in range(CH):
                        v = plsc.load_gather(rows_v, [ridx, col_idx[ch]])
                        acc[ch] = acc[ch] + w * v

        base_o = g * (CH * L)
        for ch in range(CH):
            res = jnp.where(inb, acc[ch], padv)
            plsc.store_scatter(out_v, [iota8 + (base_o + ch)], res)

    fire(0, 0)

    def body(gg, carry):
        g = gg * 2
        fire(g + 1, 1)
        consume(g, 0)

        @pl.when(gg < G // 2 - 1)
        def _():
            fire(g + 2, 0)

        consume(g + 1, 1)
        return carry

    lax.fori_loop(0, G // 2, body, 0)
    pltpu.sync_copy(out_v, out.at[wid])


@jax.jit
def _run(cp_flat, pts_t, pad):
    mesh = plsc.VectorSubcoreMesh(core_axis_name="c", subcore_axis_name="s")
    tr = functools.partial(
        pl.kernel,
        out_type=jax.ShapeDtypeStruct((NCELL, CH), jnp.float32),
        mesh=mesh,
        scratch_types=[
            pltpu.VMEM((2 * CH * TCH,), jnp.float32),
            pltpu.VMEM((2 * TCH, CH), jnp.float32),
            pltpu.SemaphoreType.DMA,
            pltpu.SemaphoreType.DMA,
            pltpu.SemaphoreType.DMA,
            pltpu.SemaphoreType.DMA,
        ],
        compiler_params=pltpu.CompilerParams(
            needs_layout_passes=False, use_tc_tiling_on_sc=False),
    )(_tr_body)
    table = tr(cp_flat)
    f = functools.partial(
        pl.kernel,
        out_type=jax.ShapeDtypeStruct((NW, PT * CH), jnp.float32),
        mesh=mesh,
        scratch_types=[
            pltpu.VMEM((PT * 3,), jnp.float32),
            pltpu.VMEM((16, 128), jnp.int32),
            pltpu.VMEM((2048, CH), jnp.float32),
            pltpu.VMEM((PT * CH,), jnp.float32),
            pltpu.VMEM((L,), jnp.float32),
            pltpu.SemaphoreType.DMA,
            pltpu.SemaphoreType.DMA,
        ],
        compiler_params=pltpu.CompilerParams(
            needs_layout_passes=False, use_tc_tiling_on_sc=False),
    )(_sc_body)
    return f(table, pts_t, pad)


def kernel(pts, control_pts, padding_value):
    n = pts.shape[0]
    pts_pad = jnp.concatenate(
        [pts.astype(jnp.float32),
         jnp.full((NPAD - n, 3), 0.5, jnp.float32)], axis=0)
    pts_t = pts_pad.reshape(NW, PT * 3)
    cp_flat = control_pts.reshape(-1)
    pad = jnp.full((L,), padding_value, jnp.float32)
    out = _run(cp_flat, pts_t, pad)
    return out.reshape(NPAD, CH)[:n]


# speedup vs baseline: 1.1073x; 1.0006x over previous
import functools

import jax
import jax.numpy as jnp
import numpy as np
from jax import lax
from jax.experimental import pallas as pl
from jax.experimental.pallas import tpu as pltpu
from jax.experimental.pallas import tpu_sc as plsc

CH = 8
GRID = 104
STEP = np.float32(0.01)
ORIGIN = np.float32(0.0) - STEP * np.float32(1.0 + 1e-8)
MAX_COORD = np.float32(102.0)

L = 16
NW = 32
G = 196
PT = G * L
NPAD = NW * PT


def _coef_list(x):
    omx = 1.0 - x
    c0 = omx * omx * omx
    c2 = (omx * omx) * (x + 1.0) * (-3.0) + 4.0
    c1 = (x - 2.0) * (x * x) * 3.0 + 4.0
    c3 = x * x * x
    return [c0 / 6.0, c1 / 6.0, c2 / 6.0, c3 / 6.0]


NCELL = GRID * GRID * GRID
CPT = NCELL // NW
TCH = 2048
TLAST = CPT - TCH


NTCH = CPT // TCH + 1


def _tr_body(cp, table, in_v, out_v, sem0, sem1, osem0, osem1):
    c = lax.axis_index("c")
    s = lax.axis_index("s")
    wid = s * 2 + c
    cellstart = wid * CPT

    iota = lax.iota(jnp.int32, L)
    sems = [sem0, sem1]
    osems = [osem0, osem1]

    def chunk_start(j):
        return cellstart + jnp.minimum(j * TCH, TLAST)

    def make_out(j, slot):
        return pltpu.make_async_copy(
            out_v.at[pl.ds(slot * TCH, TCH)],
            table.at[pl.ds(chunk_start(j), TCH)], osems[slot])

    def make_in(j, slot):
        start = chunk_start(j)
        return [
            pltpu.make_async_copy(
                cp.at[pl.ds(ch * NCELL + start, TCH)],
                in_v.at[pl.ds(slot * (CH * TCH) + ch * TCH, TCH)], sems[slot])
            for ch in range(CH)
        ]

    def fire(j, slot):
        for cpd in make_in(j, slot):
            cpd.start()

    def consume(j, slot):
        for cpd in make_in(j, slot):
            cpd.wait()

        @pl.when(j >= 2)
        def _():
            make_out(j - 2, slot).wait()

        ibase = slot * (CH * TCH)

        col8 = [jnp.full((L,), ch, jnp.int32) for ch in range(CH)]
        obase = slot * TCH

        def inter(v2, carry2):
            for u in range(2):
                v = v2 * 2 + u
                rows = iota + (obase + v * L)
                for ch in range(CH):
                    x = in_v[pl.ds(ibase + ch * TCH + v * L, L)]
                    plsc.store_scatter(out_v, [rows, col8[ch]], x)
            return carry2

        lax.fori_loop(0, TCH // L // 2, inter, 0)
        make_out(j, slot).start()

    fire(0, 0)

    def body(jj, carry):
        j = jj * 2
        fire(j + 1, 1)
        consume(j, 0)

        @pl.when(jj < NTCH // 2 - 1)
        def _():
            fire(j + 2, 0)

        consume(j + 1, 1)
        return carry

    lax.fori_loop(0, NTCH // 2, body, 0)
    make_out(NTCH - 2, 0).wait()
    make_out(NTCH - 1, 1).wait()


def _sc_body(table, pts_t, pad, out, pts_v, idx_v, rows_v, out_v, pad_v,
             sem0, sem1):
    c = lax.axis_index("c")
    s = lax.axis_index("s")
    wid = s * 2 + c

    pltpu.sync_copy(pts_t.at[wid], pts_v)
    pltpu.sync_copy(pad, pad_v)

    iota = lax.iota(jnp.int32, L)
    iota3 = iota * 3
    iota8 = iota * CH
    padv = pad_v[...]
    zero = jnp.zeros((L,), jnp.float32)
    one = jnp.full((L,), 1.0, jnp.float32)
    col_idx = [jnp.full((L,), ch, jnp.int32) for ch in range(CH)]
    sems = [sem0, sem1]

    def make_cps(slot):
        return [
            pltpu.make_async_copy(table.at[idx_v.at[slot]],
                                  rows_v.at[pl.ds(slot * 1024, 1024)],
                                  sems[slot])
        ]

    def load_p(g):
        base = g * (3 * L)
        p0 = plsc.load_gather(pts_v, [iota3 + base])
        p1 = plsc.load_gather(pts_v, [iota3 + (base + 1)])
        p2 = plsc.load_gather(pts_v, [iota3 + (base + 2)])
        p0 = (p0 - ORIGIN) / STEP
        p1 = (p1 - ORIGIN) / STEP
        p2 = (p2 - ORIGIN) / STEP
        inb = ((p0 >= 1.0) & (p1 >= 1.0) & (p2 >= 1.0)
               & (p0 < MAX_COORD) & (p1 < MAX_COORD) & (p2 < MAX_COORD))
        p0 = jnp.where(inb, p0, one)
        p1 = jnp.where(inb, p1, one)
        p2 = jnp.where(inb, p2, one)
        return p0, p1, p2, inb

    def fire(g, slot):
        p0, p1, p2, _ = load_p(g)
        i0 = p0.astype(jnp.int32)
        i1 = p1.astype(jnp.int32)
        i2 = p2.astype(jnp.int32)
        base = (i0 * GRID + i1) * GRID + i2
        for a in range(4):
            for b in range(4):
                for cc in range(4):
                    k = a * 16 + b * 4 + cc
                    off = (a - 1) * GRID * GRID + (b - 1) * GRID + (cc - 1)
                    idx_v[slot, pl.ds(k * L, L)] = base + off
        for cp in make_cps(slot):
            cp.start()

    def consume(g, slot):
        p0, p1, p2, inb = load_p(g)
        i0 = p0.astype(jnp.int32)
        i1 = p1.astype(jnp.int32)
        i2 = p2.astype(jnp.int32)
        c0s = _coef_list(p0 - i0.astype(jnp.float32))
        c1s = _coef_list(p1 - i1.astype(jnp.float32))
        c2s = _coef_list(p2 - i2.astype(jnp.float32))

        for cp in make_cps(slot):
            cp.wait()

        acc = [zero for _ in range(CH)]
        for a in range(4):
            for b in range(4):
                wab = c0s[a] * c1s[b]
                for cc in range(4):
                    k = a * 16 + b * 4 + cc
                    w = wab * c2s[cc]
                    ridx = iota + (slot * 1024 + k * L)
                    for ch in range(CH):
                        v = plsc.load_gather(rows_v, [ridx, col_idx[ch]])
                        acc[ch] = acc[ch] + w * v

        base_o = g * (CH * L)
        for ch in range(CH):
            res = jnp.where(inb, acc[ch], padv)
            plsc.store_scatter(out_v, [iota8 + (base_o + ch)], res)

    fire(0, 0)

    def body(gg, carry):
        g = gg * 2
        fire(g + 1, 1)
        consume(g, 0)

        @pl.when(gg < G // 2 - 1)
        def _():
            fire(g + 2, 0)

        consume(g + 1, 1)
        return carry

    lax.fori_loop(0, G // 2, body, 0)
    pltpu.sync_copy(out_v, out.at[wid])


@jax.jit
def _run(cp_flat, pts_t, pad):
    mesh = plsc.VectorSubcoreMesh(core_axis_name="c", subcore_axis_name="s")
    tr = functools.partial(
        pl.kernel,
        out_type=jax.ShapeDtypeStruct((NCELL, CH), jnp.float32),
        mesh=mesh,
        scratch_types=[
            pltpu.VMEM((2 * CH * TCH,), jnp.float32),
            pltpu.VMEM((2 * TCH, CH), jnp.float32),
            pltpu.SemaphoreType.DMA,
            pltpu.SemaphoreType.DMA,
            pltpu.SemaphoreType.DMA,
            pltpu.SemaphoreType.DMA,
        ],
        compiler_params=pltpu.CompilerParams(
            needs_layout_passes=False, use_tc_tiling_on_sc=False),
    )(_tr_body)
    table = tr(cp_flat)
    f = functools.partial(
        pl.kernel,
        out_type=jax.ShapeDtypeStruct((NW, PT * CH), jnp.float32),
        mesh=mesh,
        scratch_types=[
            pltpu.VMEM((PT * 3,), jnp.float32),
            pltpu.VMEM((2, 1024), jnp.int32),
            pltpu.VMEM((2048, CH), jnp.float32),
            pltpu.VMEM((PT * CH,), jnp.float32),
            pltpu.VMEM((L,), jnp.float32),
            pltpu.SemaphoreType.DMA,
            pltpu.SemaphoreType.DMA,
        ],
        compiler_params=pltpu.CompilerParams(
            needs_layout_passes=False, use_tc_tiling_on_sc=False),
    )(_sc_body)
    return f(table, pts_t, pad)


def kernel(pts, control_pts, padding_value):
    n = pts.shape[0]
    pts_pad = jnp.concatenate(
        [pts.astype(jnp.float32),
         jnp.full((NPAD - n, 3), 0.5, jnp.float32)], axis=0)
    pts_t = pts_pad.reshape(NW, PT * 3)
    cp_flat = control_pts.reshape(-1)
    pad = jnp.full((L,), padding_value, jnp.float32)
    out = _run(cp_flat, pts_t, pad)
    return out.reshape(NPAD, CH)[:n]


# speedup vs baseline: 1.1075x; 1.0003x over previous
import functools

import jax
import jax.numpy as jnp
import numpy as np
from jax import lax
from jax.experimental import pallas as pl
from jax.experimental.pallas import tpu as pltpu
from jax.experimental.pallas import tpu_sc as plsc

CH = 8
GRID = 104
STEP = np.float32(0.01)
ORIGIN = np.float32(0.0) - STEP * np.float32(1.0 + 1e-8)
MAX_COORD = np.float32(102.0)

L = 16
NW = 32
G = 196
PT = G * L
NPAD = NW * PT


def _coef_list(x):
    omx = 1.0 - x
    c0 = omx * omx * omx
    c2 = (omx * omx) * (x + 1.0) * (-3.0) + 4.0
    c1 = (x - 2.0) * (x * x) * 3.0 + 4.0
    c3 = x * x * x
    return [c0 / 6.0, c1 / 6.0, c2 / 6.0, c3 / 6.0]


NCELL = GRID * GRID * GRID
CPT = NCELL // NW
TCH = 2048
TLAST = CPT - TCH


NTCH = CPT // TCH + 1


def _tr_body(cp, table, in_v, out_v, sem0, sem1, osem0, osem1):
    c = lax.axis_index("c")
    s = lax.axis_index("s")
    wid = s * 2 + c
    cellstart = wid * CPT

    iota = lax.iota(jnp.int32, L)
    sems = [sem0, sem1]
    osems = [osem0, osem1]

    def chunk_start(j):
        return cellstart + jnp.minimum(j * TCH, TLAST)

    def make_out(j, slot):
        return pltpu.make_async_copy(
            out_v.at[pl.ds(slot * TCH, TCH)],
            table.at[pl.ds(chunk_start(j), TCH)], osems[slot])

    def make_in(j, slot):
        start = chunk_start(j)
        return [
            pltpu.make_async_copy(
                cp.at[pl.ds(ch * NCELL + start, TCH)],
                in_v.at[pl.ds(slot * (CH * TCH) + ch * TCH, TCH)], sems[slot])
            for ch in range(CH)
        ]

    def fire(j, slot):
        for cpd in make_in(j, slot):
            cpd.start()

    def consume(j, slot):
        for cpd in make_in(j, slot):
            cpd.wait()

        @pl.when(j >= 2)
        def _():
            make_out(j - 2, slot).wait()

        ibase = slot * (CH * TCH)

        col8 = [jnp.full((L,), ch, jnp.int32) for ch in range(CH)]
        obase = slot * TCH

        def inter(v2, carry2):
            for u in range(2):
                v = v2 * 2 + u
                rows = iota + (obase + v * L)
                for ch in range(CH):
                    x = in_v[pl.ds(ibase + ch * TCH + v * L, L)]
                    plsc.store_scatter(out_v, [rows, col8[ch]], x)
            return carry2

        lax.fori_loop(0, TCH // L // 2, inter, 0)
        make_out(j, slot).start()

    fire(0, 0)

    def body(jj, carry):
        j = jj * 2
        fire(j + 1, 1)
        consume(j, 0)

        @pl.when(jj < NTCH // 2 - 1)
        def _():
            fire(j + 2, 0)

        consume(j + 1, 1)
        return carry

    lax.fori_loop(0, NTCH // 2, body, 0)
    make_out(NTCH - 2, 0).wait()
    make_out(NTCH - 1, 1).wait()


def _sc_body(table, pts_t, pad, out, pts_v, idx_v, rows_v, out_v, pad_v,
             sem0, sem1):
    c = lax.axis_index("c")
    s = lax.axis_index("s")
    wid = s * 2 + c

    pltpu.sync_copy(pts_t.at[wid], pts_v)
    pltpu.sync_copy(pad, pad_v)

    iota = lax.iota(jnp.int32, L)
    iota3 = iota * 3
    iota8 = iota * CH
    padv = pad_v[...]
    zero = jnp.zeros((L,), jnp.float32)
    one = jnp.full((L,), 1.0, jnp.float32)
    col_idx = [jnp.full((L,), ch, jnp.int32) for ch in range(CH)]
    sems = [sem0, sem1]

    def make_cps(slot):
        return [
            pltpu.make_async_copy(table.at[idx_v.at[slot * 8 + j]],
                                  rows_v.at[pl.ds(slot * 1024 + j * 128, 128)],
                                  sems[slot])
            for j in range(8)
        ]

    def load_p(g):
        base = g * (3 * L)
        p0 = plsc.load_gather(pts_v, [iota3 + base])
        p1 = plsc.load_gather(pts_v, [iota3 + (base + 1)])
        p2 = plsc.load_gather(pts_v, [iota3 + (base + 2)])
        p0 = (p0 - ORIGIN) / STEP
        p1 = (p1 - ORIGIN) / STEP
        p2 = (p2 - ORIGIN) / STEP
        inb = ((p0 >= 1.0) & (p1 >= 1.0) & (p2 >= 1.0)
               & (p0 < MAX_COORD) & (p1 < MAX_COORD) & (p2 < MAX_COORD))
        p0 = jnp.where(inb, p0, one)
        p1 = jnp.where(inb, p1, one)
        p2 = jnp.where(inb, p2, one)
        return p0, p1, p2, inb

    def fire(g, slot):
        p0, p1, p2, _ = load_p(g)
        i0 = p0.astype(jnp.int32)
        i1 = p1.astype(jnp.int32)
        i2 = p2.astype(jnp.int32)
        base = (i0 * GRID + i1) * GRID + i2
        for a in range(4):
            for b in range(4):
                for cc in range(4):
                    k = a * 16 + b * 4 + cc
                    off = (a - 1) * GRID * GRID + (b - 1) * GRID + (cc - 1)
                    idx_v[slot * 8 + k // 8, pl.ds((k % 8) * L, L)] = base + off
        for cp in make_cps(slot):
            cp.start()

    def consume(g, slot):
        p0, p1, p2, inb = load_p(g)
        i0 = p0.astype(jnp.int32)
        i1 = p1.astype(jnp.int32)
        i2 = p2.astype(jnp.int32)
        c0s = _coef_list(p0 - i0.astype(jnp.float32))
        c1s = _coef_list(p1 - i1.astype(jnp.float32))
        c2s = _coef_list(p2 - i2.astype(jnp.float32))

        for cp in make_cps(slot):
            cp.wait()

        acc = [zero for _ in range(CH)]
        for a in range(4):
            for b in range(4):
                wab = c0s[a] * c1s[b]
                for cc in range(4):
                    k = a * 16 + b * 4 + cc
                    w = wab * c2s[cc]
                    ridx = iota + (slot * 1024 + k * L)
                    for ch in range(CH):
                        v = plsc.load_gather(rows_v, [ridx, col_idx[ch]])
                        acc[ch] = acc[ch] + w * v

        base_o = g * (CH * L)
        for ch in range(CH):
            res = jnp.where(inb, acc[ch], padv)
            plsc.store_scatter(out_v, [iota8 + (base_o + ch)], res)

    fire(0, 0)

    def body(gg, carry):
        g = gg * 2
        fire(g + 1, 1)
        consume(g, 0)

        @pl.when(gg < G // 2 - 1)
        def _():
            fire(g + 2, 0)

        consume(g + 1, 1)
        return carry

    lax.fori_loop(0, G // 2, body, 0)
    pltpu.sync_copy(out_v, out.at[wid])


@jax.jit
def _run(cp_flat, pts_t, pad):
    mesh = plsc.VectorSubcoreMesh(core_axis_name="c", subcore_axis_name="s")
    tr = functools.partial(
        pl.kernel,
        out_type=jax.ShapeDtypeStruct((NCELL, CH), jnp.float32),
        mesh=mesh,
        scratch_types=[
            pltpu.VMEM((2 * CH * TCH,), jnp.float32),
            pltpu.VMEM((2 * TCH, CH), jnp.float32),
            pltpu.SemaphoreType.DMA,
            pltpu.SemaphoreType.DMA,
            pltpu.SemaphoreType.DMA,
            pltpu.SemaphoreType.DMA,
        ],
        compiler_params=pltpu.CompilerParams(
            needs_layout_passes=False, use_tc_tiling_on_sc=False),
    )(_tr_body)
    table = tr(cp_flat)
    f = functools.partial(
        pl.kernel,
        out_type=jax.ShapeDtypeStruct((NW, PT * CH), jnp.float32),
        mesh=mesh,
        scratch_types=[
            pltpu.VMEM((PT * 3,), jnp.float32),
            pltpu.VMEM((16, 128), jnp.int32),
            pltpu.VMEM((2048, CH), jnp.float32),
            pltpu.VMEM((PT * CH,), jnp.float32),
            pltpu.VMEM((L,), jnp.float32),
            pltpu.SemaphoreType.DMA,
            pltpu.SemaphoreType.DMA,
        ],
        compiler_params=pltpu.CompilerParams(
            needs_layout_passes=False, use_tc_tiling_on_sc=False),
    )(_sc_body)
    return f(table, pts_t, pad)


def kernel(pts, control_pts, padding_value):
    n = pts.shape[0]
    pts_pad = jnp.concatenate(
        [pts.astype(jnp.float32),
         jnp.full((NPAD - n, 3), 0.5, jnp.float32)], axis=0)
    pts_t = pts_pad.reshape(NW, PT * 3)
    cp_flat = control_pts.reshape(-1)
    pad = jnp.full((L,), padding_value, jnp.float32)
    out = _run(cp_flat, pts_t, pad)
    return out.reshape(NPAD, CH)[:n]


# speedup vs baseline: 1.1083x; 1.0007x over previous
import functools

import jax
import jax.numpy as jnp
import numpy as np
from jax import lax
from jax.experimental import pallas as pl
from jax.experimental.pallas import tpu as pltpu
from jax.experimental.pallas import tpu_sc as plsc

CH = 8
GRID = 104
STEP = np.float32(0.01)
ORIGIN = np.float32(0.0) - STEP * np.float32(1.0 + 1e-8)
MAX_COORD = np.float32(102.0)

L = 16
NW = 32
G = 196
PT = G * L
NPAD = NW * PT


def _coef_list(x):
    omx = 1.0 - x
    c0 = omx * omx * omx
    c2 = (omx * omx) * (x + 1.0) * (-3.0) + 4.0
    c1 = (x - 2.0) * (x * x) * 3.0 + 4.0
    c3 = x * x * x
    return [c0 / 6.0, c1 / 6.0, c2 / 6.0, c3 / 6.0]


NCELL = GRID * GRID * GRID
CPT = NCELL // NW
TCH = 2048
TLAST = CPT - TCH


NTCH = CPT // TCH + 1


def _tr_body(cp, table, in_v, out_v, sem0, sem1, osem0, osem1):
    c = lax.axis_index("c")
    s = lax.axis_index("s")
    wid = s * 2 + c
    cellstart = wid * CPT

    iota = lax.iota(jnp.int32, L)
    sems = [sem0, sem1]
    osems = [osem0, osem1]

    def chunk_start(j):
        return cellstart + jnp.minimum(j * TCH, TLAST)

    def make_out(j, slot):
        return pltpu.make_async_copy(
            out_v.at[pl.ds(slot * TCH, TCH)],
            table.at[pl.ds(chunk_start(j), TCH)], osems[slot])

    def make_in(j, slot):
        start = chunk_start(j)
        return [
            pltpu.make_async_copy(
                cp.at[pl.ds(ch * NCELL + start, TCH)],
                in_v.at[pl.ds(slot * (CH * TCH) + ch * TCH, TCH)], sems[slot])
            for ch in range(CH)
        ]

    def fire(j, slot):
        for cpd in make_in(j, slot):
            cpd.start()

    def consume(j, slot):
        for cpd in make_in(j, slot):
            cpd.wait()

        @pl.when(j >= 2)
        def _():
            make_out(j - 2, slot).wait()

        ibase = slot * (CH * TCH)

        col8 = [jnp.full((L,), ch, jnp.int32) for ch in range(CH)]
        obase = slot * TCH

        def inter(v2, carry2):
            for u in range(4):
                v = v2 * 4 + u
                rows = iota + (obase + v * L)
                for ch in range(CH):
                    x = in_v[pl.ds(ibase + ch * TCH + v * L, L)]
                    plsc.store_scatter(out_v, [rows, col8[ch]], x)
            return carry2

        lax.fori_loop(0, TCH // L // 4, inter, 0)
        make_out(j, slot).start()

    fire(0, 0)

    def body(jj, carry):
        j = jj * 2
        fire(j + 1, 1)
        consume(j, 0)

        @pl.when(jj < NTCH // 2 - 1)
        def _():
            fire(j + 2, 0)

        consume(j + 1, 1)
        return carry

    lax.fori_loop(0, NTCH // 2, body, 0)
    make_out(NTCH - 2, 0).wait()
    make_out(NTCH - 1, 1).wait()


def _sc_body(table, pts_t, pad, out, pts_v, idx_v, rows_v, out_v, pad_v,
             sem0, sem1):
    c = lax.axis_index("c")
    s = lax.axis_index("s")
    wid = s * 2 + c

    pltpu.sync_copy(pts_t.at[wid], pts_v)
    pltpu.sync_copy(pad, pad_v)

    iota = lax.iota(jnp.int32, L)
    iota3 = iota * 3
    iota8 = iota * CH
    padv = pad_v[...]
    zero = jnp.zeros((L,), jnp.float32)
    one = jnp.full((L,), 1.0, jnp.float32)
    col_idx = [jnp.full((L,), ch, jnp.int32) for ch in range(CH)]
    sems = [sem0, sem1]

    def make_cps(slot):
        return [
            pltpu.make_async_copy(table.at[idx_v.at[slot * 8 + j]],
                                  rows_v.at[pl.ds(slot * 1024 + j * 128, 128)],
                                  sems[slot])
            for j in range(8)
        ]

    def load_p(g):
        base = g * (3 * L)
        p0 = plsc.load_gather(pts_v, [iota3 + base])
        p1 = plsc.load_gather(pts_v, [iota3 + (base + 1)])
        p2 = plsc.load_gather(pts_v, [iota3 + (base + 2)])
        p0 = (p0 - ORIGIN) / STEP
        p1 = (p1 - ORIGIN) / STEP
        p2 = (p2 - ORIGIN) / STEP
        inb = ((p0 >= 1.0) & (p1 >= 1.0) & (p2 >= 1.0)
               & (p0 < MAX_COORD) & (p1 < MAX_COORD) & (p2 < MAX_COORD))
        p0 = jnp.where(inb, p0, one)
        p1 = jnp.where(inb, p1, one)
        p2 = jnp.where(inb, p2, one)
        return p0, p1, p2, inb

    def fire(g, slot):
        p0, p1, p2, _ = load_p(g)
        i0 = p0.astype(jnp.int32)
        i1 = p1.astype(jnp.int32)
        i2 = p2.astype(jnp.int32)
        base = (i0 * GRID + i1) * GRID + i2
        for a in range(4):
            for b in range(4):
                for cc in range(4):
                    k = a * 16 + b * 4 + cc
                    off = (a - 1) * GRID * GRID + (b - 1) * GRID + (cc - 1)
                    idx_v[slot * 8 + k // 8, pl.ds((k % 8) * L, L)] = base + off
        for cp in make_cps(slot):
            cp.start()

    def consume(g, slot):
        p0, p1, p2, inb = load_p(g)
        i0 = p0.astype(jnp.int32)
        i1 = p1.astype(jnp.int32)
        i2 = p2.astype(jnp.int32)
        c0s = _coef_list(p0 - i0.astype(jnp.float32))
        c1s = _coef_list(p1 - i1.astype(jnp.float32))
        c2s = _coef_list(p2 - i2.astype(jnp.float32))

        for cp in make_cps(slot):
            cp.wait()

        acc = [zero for _ in range(CH)]
        for a in range(4):
            for b in range(4):
                wab = c0s[a] * c1s[b]
                for cc in range(4):
                    k = a * 16 + b * 4 + cc
                    w = wab * c2s[cc]
                    ridx = iota + (slot * 1024 + k * L)
                    for ch in range(CH):
                        v = plsc.load_gather(rows_v, [ridx, col_idx[ch]])
                        acc[ch] = acc[ch] + w * v

        base_o = g * (CH * L)
        for ch in range(CH):
            res = jnp.where(inb, acc[ch], padv)
            plsc.store_scatter(out_v, [iota8 + (base_o + ch)], res)

    fire(0, 0)

    def body(gg, carry):
        g = gg * 2
        fire(g + 1, 1)
        consume(g, 0)

        @pl.when(gg < G // 2 - 1)
        def _():
            fire(g + 2, 0)

        consume(g + 1, 1)
        return carry

    lax.fori_loop(0, G // 2, body, 0)
    pltpu.sync_copy(out_v, out.at[wid])


@jax.jit
def _run(cp_flat, pts_t, pad):
    mesh = plsc.VectorSubcoreMesh(core_axis_name="c", subcore_axis_name="s")
    tr = functools.partial(
        pl.kernel,
        out_type=jax.ShapeDtypeStruct((NCELL, CH), jnp.float32),
        mesh=mesh,
        scratch_types=[
            pltpu.VMEM((2 * CH * TCH,), jnp.float32),
            pltpu.VMEM((2 * TCH, CH), jnp.float32),
            pltpu.SemaphoreType.DMA,
            pltpu.SemaphoreType.DMA,
            pltpu.SemaphoreType.DMA,
            pltpu.SemaphoreType.DMA,
        ],
        compiler_params=pltpu.CompilerParams(
            needs_layout_passes=False, use_tc_tiling_on_sc=False),
    )(_tr_body)
    table = tr(cp_flat)
    f = functools.partial(
        pl.kernel,
        out_type=jax.ShapeDtypeStruct((NW, PT * CH), jnp.float32),
        mesh=mesh,
        scratch_types=[
            pltpu.VMEM((PT * 3,), jnp.float32),
            pltpu.VMEM((16, 128), jnp.int32),
            pltpu.VMEM((2048, CH), jnp.float32),
            pltpu.VMEM((PT * CH,), jnp.float32),
            pltpu.VMEM((L,), jnp.float32),
            pltpu.SemaphoreType.DMA,
            pltpu.SemaphoreType.DMA,
        ],
        compiler_params=pltpu.CompilerParams(
            needs_layout_passes=False, use_tc_tiling_on_sc=False),
    )(_sc_body)
    return f(table, pts_t, pad)


def kernel(pts, control_pts, padding_value):
    n = pts.shape[0]
    pts_pad = jnp.concatenate(
        [pts.astype(jnp.float32),
         jnp.full((NPAD - n, 3), 0.5, jnp.float32)], axis=0)
    pts_t = pts_pad.reshape(NW, PT * 3)
    cp_flat = control_pts.reshape(-1)
    pad = jnp.full((L,), padding_value, jnp.float32)
    out = _run(cp_flat, pts_t, pad)
    return out.reshape(NPAD, CH)[:n]
